# baseline (speedup 1.0000x reference)
import functools

import numpy as np
import jax
import jax.numpy as jnp
from jax import lax
from jax.experimental import pallas as pl
from jax.experimental.pallas import tpu as pltpu
from jax.experimental.pallas import tpu_sc as plsc

_RES = 384
_NPTS = _RES * _RES // 4
_BATCH = 8

_NW = 32
_NTEAM = 4
_NPOS = _NW // _NTEAM
_ROWS_PER_W = _RES // _NPOS
_PLANE = _ROWS_PER_W * _RES
_SLAB = 5 * _PLANE
_PTS_PER_TEAM = _NPTS // _NTEAM

_jj = np.arange(_RES, dtype=np.float64)
_ph = np.exp(2j * np.pi * np.outer(_jj, _jj) / _RES)
_sgn = (-1.0) ** (_jj[:, None] + _jj[None, :])
_E = _sgn * _ph
_C_NP = np.ascontiguousarray(_E.real)
_S_NP = np.ascontiguousarray(_E.imag)
_CPS_NP = _C_NP + _S_NP

_OFFS = ((0, 1), (1, 0), (1, 1), (1, -1))


_UNROLL = 4


def _k5_body(traj_hbm, zeros_hbm, out_hbm, tx_v, ty_v, slab_v):
    wid = lax.axis_index("s") * 2 + lax.axis_index("c")
    team = wid % _NTEAM
    pos = wid // _NTEAM
    row_lo = pos * _ROWS_PER_W
    row_hi = row_lo + _ROWS_PER_W
    samp_lo = team * _PTS_PER_TEAM

    pltpu.sync_copy(traj_hbm.at[0, pl.ds(samp_lo, _PTS_PER_TEAM)], tx_v)
    pltpu.sync_copy(traj_hbm.at[1, pl.ds(samp_lo, _PTS_PER_TEAM)], ty_v)
    pltpu.sync_copy(zeros_hbm, slab_v)

    cx = jnp.float32((_RES - 1) / _RES)
    c0 = jnp.float32((_RES - 1) / 2.0)
    fmax = jnp.float32(_RES - 1)

    def _group(g):
        s = pl.ds(g * 16, 16)
        ty = ty_v[s]
        gy = jnp.minimum(jnp.maximum(ty * cx + c0, 0.0), fmax)
        yi = gy.astype(jnp.int32)
        in0 = (yi >= row_lo) & (yi < row_hi)
        y1i = yi + 1
        in1 = (y1i >= row_lo) & (y1i < row_hi)

        tx = tx_v[s]
        gx = jnp.minimum(jnp.maximum(tx * cx + c0, 0.0), fmax)
        xi = gx.astype(jnp.int32)
        wx1 = gx - xi.astype(jnp.float32)
        wy1 = gy - yi.astype(jnp.float32)
        wx0 = 1.0 - wx1
        wy0 = 1.0 - wy1
        w00 = wy0 * wx0
        w01 = wy0 * wx1
        w10 = wy1 * wx0
        w11 = wy1 * wx1

        base = (yi - row_lo) * _RES + xi

        plsc.addupdate_scatter(slab_v, [base], w00 * w00, mask=in0)
        plsc.addupdate_scatter(slab_v, [base + _PLANE], w00 * w01, mask=in0)
        plsc.addupdate_scatter(slab_v, [base + 2 * _PLANE], w00 * w10, mask=in0)
        plsc.addupdate_scatter(slab_v, [base + 3 * _PLANE], w00 * w11, mask=in0)
        plsc.addupdate_scatter(slab_v, [base + 1], w01 * w01, mask=in0)
        plsc.addupdate_scatter(slab_v, [base + 1 + 4 * _PLANE], w01 * w10, mask=in0)
        plsc.addupdate_scatter(slab_v, [base + 1 + 2 * _PLANE], w01 * w11, mask=in0)
        plsc.addupdate_scatter(slab_v, [base + _RES], w10 * w10, mask=in1)
        plsc.addupdate_scatter(slab_v, [base + _RES + _PLANE], w10 * w11, mask=in1)
        plsc.addupdate_scatter(slab_v, [base + _RES + 1], w11 * w11, mask=in1)

    def _step(i, carry):
        for j in range(_UNROLL):
            _group(i * _UNROLL + j)
        return carry

    lax.fori_loop(0, _PTS_PER_TEAM // 16 // _UNROLL, _step, 0)

    for p in range(5):
        pltpu.sync_copy(
            slab_v.at[pl.ds(p * _PLANE, _PLANE)],
            out_hbm.at[team, pl.ds(p * _RES * _RES + row_lo * _RES, _PLANE)],
        )


_k5_call = functools.partial(
    pl.kernel,
    mesh=plsc.VectorSubcoreMesh(core_axis_name="c", subcore_axis_name="s"),
    out_type=jax.ShapeDtypeStruct((_NTEAM, 5 * _RES * _RES), jnp.float32),
    scratch_types=[
        pltpu.VMEM((_PTS_PER_TEAM,), jnp.float32),
        pltpu.VMEM((_PTS_PER_TEAM,), jnp.float32),
        pltpu.VMEM((_SLAB,), jnp.float32),
    ],
    compiler_params=pltpu.CompilerParams(needs_layout_passes=False),
)(_k5_body)


def _roll2(a, dy, dx):
    if dy:
        a = jnp.concatenate([a[-dy % _RES:, :], a[:-dy % _RES, :]], axis=0)
    if dx:
        a = jnp.concatenate([a[:, -dx % _RES:], a[:, :-dx % _RES]], axis=1)
    return a


def _tc_body(ksp_ref, k5_ref, c_ref, s_ref, cps_ref, out_ref, ks_ref):
    b = pl.program_id(0)

    @pl.when(b == 0)
    def _sum_parts():
        for p in range(5):
            ks_ref[p] = (k5_ref[0, p] + k5_ref[1, p]) + (k5_ref[2, p] + k5_ref[3, p])

    zr = ksp_ref[0, 0]
    zi = ksp_ref[0, 1]

    def stencil(z):
        g = ks_ref[0] * z
        for p, (dy, dx) in enumerate(_OFFS):
            kd = ks_ref[1 + p]
            g = g + kd * _roll2(z, -dy, -dx) + _roll2(kd * z, dy, dx)
        return g

    gr = stencil(zr)
    gi = stencil(zi)

    cm = c_ref[...]
    sm = s_ref[...]
    cps = cps_ref[...]

    def mm(a, b):
        return jnp.dot(a, b, preferred_element_type=jnp.float32)

    m1 = mm(cm, gr)
    m2 = mm(sm, gi)
    m3 = mm(cps, gr + gi)
    tr = m1 - m2
    ti = m3 - m1 - m2
    n1 = mm(tr, cm)
    n2 = mm(ti, sm)
    n3 = mm(tr + ti, cps)
    out_ref[0, 0, 0] = n1 - n2
    out_ref[0, 0, 1] = n3 - n1 - n2


def _tc_call(ksp, k5_parts, cm, sm, cps):
    return pl.pallas_call(
        _tc_body,
        grid=(_BATCH,),
        in_specs=[
            pl.BlockSpec((1, 2, _RES, _RES), lambda b: (b, 0, 0, 0)),
            pl.BlockSpec((_NTEAM, 5, _RES, _RES), lambda b: (0, 0, 0, 0)),
            pl.BlockSpec((_RES, _RES), lambda b: (0, 0)),
            pl.BlockSpec((_RES, _RES), lambda b: (0, 0)),
            pl.BlockSpec((_RES, _RES), lambda b: (0, 0)),
        ],
        out_specs=pl.BlockSpec((1, 1, 2, _RES, _RES), lambda b: (b, 0, 0, 0, 0)),
        out_shape=jax.ShapeDtypeStruct((_BATCH, 1, 2, _RES, _RES), jnp.float32),
        scratch_shapes=[pltpu.VMEM((5, _RES, _RES), jnp.float32)],
        compiler_params=pltpu.CompilerParams(
            dimension_semantics=("arbitrary",),
        ),
    )(ksp, k5_parts, cm, sm, cps)


def kernel(k_space_input, trajectory):
    ksp = jnp.transpose(k_space_input, (0, 1, 4, 2, 3))[:, 0]

    traj_re = (trajectory.reshape(16, _NPTS // 16, 2)
               .transpose(2, 1, 0)
               .reshape(2, _NPTS))
    k5_parts = _k5_call(traj_re, jnp.zeros((_SLAB,), jnp.float32))
    k5_parts = k5_parts.reshape(_NTEAM, 5, _RES, _RES)

    cm = jnp.asarray(_C_NP, jnp.float32)
    sm = jnp.asarray(_S_NP, jnp.float32)
    cps = jnp.asarray(_CPS_NP, jnp.float32)

    return _tc_call(ksp, k5_parts, cm, sm, cps)


# speedup vs baseline: 1.0683x; 1.0683x over previous
import functools

import numpy as np
import jax
import jax.numpy as jnp
from jax import lax
from jax.experimental import pallas as pl
from jax.experimental.pallas import tpu as pltpu
from jax.experimental.pallas import tpu_sc as plsc

_RES = 384
_NPTS = _RES * _RES // 4
_BATCH = 8

_NW = 32
_NTEAM = 4
_NPOS = _NW // _NTEAM
_ROWS_PER_W = _RES // _NPOS
_PLANE = _ROWS_PER_W * _RES
_SLAB = 5 * _PLANE
_PTS_PER_TEAM = _NPTS // _NTEAM

_jj = np.arange(_RES, dtype=np.float64)
_ph = np.exp(2j * np.pi * np.outer(_jj, _jj) / _RES)
_sgn = (-1.0) ** (_jj[:, None] + _jj[None, :])
_E = _sgn * _ph
_C_NP = np.ascontiguousarray(_E.real)
_S_NP = np.ascontiguousarray(_E.imag)
_CPS_NP = _C_NP + _S_NP

_OFFS = ((0, 1), (1, 0), (1, 1), (1, -1))


_UNROLL = 4


def _k5_body(traj_hbm, out_hbm, tx_v, ty_v, slab_v):
    wid = lax.axis_index("s") * 2 + lax.axis_index("c")
    team = wid % _NTEAM
    pos = wid // _NTEAM
    row_lo = pos * _ROWS_PER_W
    row_hi = row_lo + _ROWS_PER_W
    samp_lo = team * _PTS_PER_TEAM

    pltpu.sync_copy(traj_hbm.at[0, pl.ds(samp_lo, _PTS_PER_TEAM)], tx_v)
    pltpu.sync_copy(traj_hbm.at[1, pl.ds(samp_lo, _PTS_PER_TEAM)], ty_v)

    zv = jnp.zeros((16,), jnp.float32)

    def _zero(i, carry):
        for j in range(8):
            slab_v[pl.ds((i * 8 + j) * 16, 16)] = zv
        return carry

    lax.fori_loop(0, _SLAB // 16 // 8, _zero, 0)

    cx = jnp.float32((_RES - 1) / _RES)
    c0 = jnp.float32((_RES - 1) / 2.0)
    fmax = jnp.float32(_RES - 1)

    def _group(g):
        s = pl.ds(g * 16, 16)
        ty = ty_v[s]
        gy = jnp.minimum(jnp.maximum(ty * cx + c0, 0.0), fmax)
        yi = gy.astype(jnp.int32)
        in0 = (yi >= row_lo) & (yi < row_hi)
        y1i = yi + 1
        in1 = (y1i >= row_lo) & (y1i < row_hi)

        tx = tx_v[s]
        gx = jnp.minimum(jnp.maximum(tx * cx + c0, 0.0), fmax)
        xi = gx.astype(jnp.int32)
        wx1 = gx - xi.astype(jnp.float32)
        wy1 = gy - yi.astype(jnp.float32)
        wx0 = 1.0 - wx1
        wy0 = 1.0 - wy1
        w00 = wy0 * wx0
        w01 = wy0 * wx1
        w10 = wy1 * wx0
        w11 = wy1 * wx1

        base = (yi - row_lo) * _RES + xi

        plsc.addupdate_scatter(slab_v, [base], w00 * w00, mask=in0)
        plsc.addupdate_scatter(slab_v, [base + _PLANE], w00 * w01, mask=in0)
        plsc.addupdate_scatter(slab_v, [base + 2 * _PLANE], w00 * w10, mask=in0)
        plsc.addupdate_scatter(slab_v, [base + 3 * _PLANE], w00 * w11, mask=in0)
        plsc.addupdate_scatter(slab_v, [base + 1], w01 * w01, mask=in0)
        plsc.addupdate_scatter(slab_v, [base + 1 + 4 * _PLANE], w01 * w10, mask=in0)
        plsc.addupdate_scatter(slab_v, [base + 1 + 2 * _PLANE], w01 * w11, mask=in0)
        plsc.addupdate_scatter(slab_v, [base + _RES], w10 * w10, mask=in1)
        plsc.addupdate_scatter(slab_v, [base + _RES + _PLANE], w10 * w11, mask=in1)
        plsc.addupdate_scatter(slab_v, [base + _RES + 1], w11 * w11, mask=in1)

    def _step(i, carry):
        for j in range(_UNROLL):
            _group(i * _UNROLL + j)
        return carry

    lax.fori_loop(0, _PTS_PER_TEAM // 16 // _UNROLL, _step, 0)

    for p in range(5):
        pltpu.sync_copy(
            slab_v.at[pl.ds(p * _PLANE, _PLANE)],
            out_hbm.at[team, pl.ds(p * _RES * _RES + row_lo * _RES, _PLANE)],
        )


_k5_call = functools.partial(
    pl.kernel,
    mesh=plsc.VectorSubcoreMesh(core_axis_name="c", subcore_axis_name="s"),
    out_type=jax.ShapeDtypeStruct((_NTEAM, 5 * _RES * _RES), jnp.float32),
    scratch_types=[
        pltpu.VMEM((_PTS_PER_TEAM,), jnp.float32),
        pltpu.VMEM((_PTS_PER_TEAM,), jnp.float32),
        pltpu.VMEM((_SLAB,), jnp.float32),
    ],
    compiler_params=pltpu.CompilerParams(needs_layout_passes=False),
)(_k5_body)


def _roll2(a, dy, dx):
    if dy:
        a = jnp.concatenate([a[-dy % _RES:, :], a[:-dy % _RES, :]], axis=0)
    if dx:
        a = jnp.concatenate([a[:, -dx % _RES:], a[:, :-dx % _RES]], axis=1)
    return a


def _tc_body(ksp_ref, k5_ref, c_ref, s_ref, cps_ref, out_ref, ks_ref):
    b = pl.program_id(0)

    @pl.when(b == 0)
    def _sum_parts():
        for p in range(5):
            ks_ref[p] = (k5_ref[0, p] + k5_ref[1, p]) + (k5_ref[2, p] + k5_ref[3, p])

    zr = ksp_ref[0, 0]
    zi = ksp_ref[0, 1]

    def stencil(z):
        g = ks_ref[0] * z
        for p, (dy, dx) in enumerate(_OFFS):
            kd = ks_ref[1 + p]
            g = g + kd * _roll2(z, -dy, -dx) + _roll2(kd * z, dy, dx)
        return g

    gr = stencil(zr)
    gi = stencil(zi)

    cm = c_ref[...]
    sm = s_ref[...]
    cps = cps_ref[...]

    def mm(a, b):
        return jnp.dot(a, b, preferred_element_type=jnp.float32)

    m1 = mm(cm, gr)
    m2 = mm(sm, gi)
    m3 = mm(cps, gr + gi)
    tr = m1 - m2
    ti = m3 - m1 - m2
    n1 = mm(tr, cm)
    n2 = mm(ti, sm)
    n3 = mm(tr + ti, cps)
    out_ref[0, 0, 0] = n1 - n2
    out_ref[0, 0, 1] = n3 - n1 - n2


def _tc_call(ksp, k5_parts, cm, sm, cps):
    return pl.pallas_call(
        _tc_body,
        grid=(_BATCH,),
        in_specs=[
            pl.BlockSpec((1, 2, _RES, _RES), lambda b: (b, 0, 0, 0)),
            pl.BlockSpec((_NTEAM, 5, _RES, _RES), lambda b: (0, 0, 0, 0)),
            pl.BlockSpec((_RES, _RES), lambda b: (0, 0)),
            pl.BlockSpec((_RES, _RES), lambda b: (0, 0)),
            pl.BlockSpec((_RES, _RES), lambda b: (0, 0)),
        ],
        out_specs=pl.BlockSpec((1, 1, 2, _RES, _RES), lambda b: (b, 0, 0, 0, 0)),
        out_shape=jax.ShapeDtypeStruct((_BATCH, 1, 2, _RES, _RES), jnp.float32),
        scratch_shapes=[pltpu.VMEM((5, _RES, _RES), jnp.float32)],
        compiler_params=pltpu.CompilerParams(
            dimension_semantics=("arbitrary",),
        ),
    )(ksp, k5_parts, cm, sm, cps)


def kernel(k_space_input, trajectory):
    ksp = jnp.transpose(k_space_input, (0, 1, 4, 2, 3))[:, 0]

    traj_re = (trajectory.reshape(16, _NPTS // 16, 2)
               .transpose(2, 1, 0)
               .reshape(2, _NPTS))
    k5_parts = _k5_call(traj_re)
    k5_parts = k5_parts.reshape(_NTEAM, 5, _RES, _RES)

    cm = jnp.asarray(_C_NP, jnp.float32)
    sm = jnp.asarray(_S_NP, jnp.float32)
    cps = jnp.asarray(_CPS_NP, jnp.float32)

    return _tc_call(ksp, k5_parts, cm, sm, cps)


# speedup vs baseline: 1.0748x; 1.0060x over previous
import functools

import numpy as np
import jax
import jax.numpy as jnp
from jax import lax
from jax.experimental import pallas as pl
from jax.experimental.pallas import tpu as pltpu
from jax.experimental.pallas import tpu_sc as plsc

_RES = 384
_NPTS = _RES * _RES // 4
_BATCH = 8

_NW = 32
_NTEAM = 4
_NPOS = _NW // _NTEAM
_ROWS_PER_W = _RES // _NPOS
_PLANE = _ROWS_PER_W * _RES
_SLAB = 5 * _PLANE
_PTS_PER_TEAM = _NPTS // _NTEAM

_jj = np.arange(_RES, dtype=np.float64)
_ph = np.exp(2j * np.pi * np.outer(_jj, _jj) / _RES)
_sgn = (-1.0) ** (_jj[:, None] + _jj[None, :])
_E = _sgn * _ph
_C_NP = np.ascontiguousarray(_E.real)
_S_NP = np.ascontiguousarray(_E.imag)
_CPS_NP = _C_NP + _S_NP

_OFFS = ((0, 1), (1, 0), (1, 1), (1, -1))


_UNROLL = 8


def _k5_body(traj_hbm, out_hbm, txy_v, slab_v):
    wid = lax.axis_index("s") * 2 + lax.axis_index("c")
    team = wid % _NTEAM
    pos = wid // _NTEAM
    row_lo = pos * _ROWS_PER_W
    row_hi = row_lo + _ROWS_PER_W
    samp_lo = team * _PTS_PER_TEAM

    pltpu.sync_copy(traj_hbm.at[:, pl.ds(samp_lo, _PTS_PER_TEAM)], txy_v)

    zv = jnp.zeros((16,), jnp.float32)

    def _zero(i, carry):
        for j in range(8):
            slab_v[pl.ds((i * 8 + j) * 16, 16)] = zv
        return carry

    lax.fori_loop(0, _SLAB // 16 // 8, _zero, 0)

    cx = jnp.float32((_RES - 1) / _RES)
    c0 = jnp.float32((_RES - 1) / 2.0)
    fmax = jnp.float32(_RES - 1)

    def _group(g):
        s = pl.ds(g * 16, 16)
        ty = txy_v[1, s]
        gy = jnp.minimum(jnp.maximum(ty * cx + c0, 0.0), fmax)
        yi = gy.astype(jnp.int32)
        in0 = (yi >= row_lo) & (yi < row_hi)
        y1i = yi + 1
        in1 = (y1i >= row_lo) & (y1i < row_hi)

        tx = txy_v[0, s]
        gx = jnp.minimum(jnp.maximum(tx * cx + c0, 0.0), fmax)
        xi = gx.astype(jnp.int32)
        wx1 = gx - xi.astype(jnp.float32)
        wy1 = gy - yi.astype(jnp.float32)
        wx0 = 1.0 - wx1
        wy0 = 1.0 - wy1
        w00 = wy0 * wx0
        w01 = wy0 * wx1
        w10 = wy1 * wx0
        w11 = wy1 * wx1

        base = (yi - row_lo) * _RES + xi

        plsc.addupdate_scatter(slab_v, [base], w00 * w00, mask=in0)
        plsc.addupdate_scatter(slab_v, [base + _PLANE], w00 * w01, mask=in0)
        plsc.addupdate_scatter(slab_v, [base + 2 * _PLANE], w00 * w10, mask=in0)
        plsc.addupdate_scatter(slab_v, [base + 3 * _PLANE], w00 * w11, mask=in0)
        plsc.addupdate_scatter(slab_v, [base + 1], w01 * w01, mask=in0)
        plsc.addupdate_scatter(slab_v, [base + 1 + 4 * _PLANE], w01 * w10, mask=in0)
        plsc.addupdate_scatter(slab_v, [base + 1 + 2 * _PLANE], w01 * w11, mask=in0)
        plsc.addupdate_scatter(slab_v, [base + _RES], w10 * w10, mask=in1)
        plsc.addupdate_scatter(slab_v, [base + _RES + _PLANE], w10 * w11, mask=in1)
        plsc.addupdate_scatter(slab_v, [base + _RES + 1], w11 * w11, mask=in1)

    def _step(i, carry):
        for j in range(_UNROLL):
            _group(i * _UNROLL + j)
        return carry

    lax.fori_loop(0, _PTS_PER_TEAM // 16 // _UNROLL, _step, 0)

    for p in range(5):
        pltpu.sync_copy(
            slab_v.at[pl.ds(p * _PLANE, _PLANE)],
            out_hbm.at[team, pl.ds(p * _RES * _RES + row_lo * _RES, _PLANE)],
        )


_k5_call = functools.partial(
    pl.kernel,
    mesh=plsc.VectorSubcoreMesh(core_axis_name="c", subcore_axis_name="s"),
    out_type=jax.ShapeDtypeStruct((_NTEAM, 5 * _RES * _RES), jnp.float32),
    scratch_types=[
        pltpu.VMEM((2, _PTS_PER_TEAM), jnp.float32),
        pltpu.VMEM((_SLAB,), jnp.float32),
    ],
    compiler_params=pltpu.CompilerParams(needs_layout_passes=False),
)(_k5_body)


def _roll2(a, dy, dx):
    if dy:
        a = jnp.concatenate([a[-dy % _RES:, :], a[:-dy % _RES, :]], axis=0)
    if dx:
        a = jnp.concatenate([a[:, -dx % _RES:], a[:, :-dx % _RES]], axis=1)
    return a


def _tc_body(ksp_ref, k5_ref, c_ref, s_ref, cps_ref, out_ref, ks_ref):
    b = pl.program_id(0)

    @pl.when(b == 0)
    def _sum_parts():
        for p in range(5):
            ks_ref[p] = (k5_ref[0, p] + k5_ref[1, p]) + (k5_ref[2, p] + k5_ref[3, p])

    zr = ksp_ref[0, 0]
    zi = ksp_ref[0, 1]

    def stencil(z):
        g = ks_ref[0] * z
        for p, (dy, dx) in enumerate(_OFFS):
            kd = ks_ref[1 + p]
            g = g + kd * _roll2(z, -dy, -dx) + _roll2(kd * z, dy, dx)
        return g

    gr = stencil(zr)
    gi = stencil(zi)

    cm = c_ref[...]
    sm = s_ref[...]
    cps = cps_ref[...]

    def mm(a, b):
        return jnp.dot(a, b, preferred_element_type=jnp.float32)

    m1 = mm(cm, gr)
    m2 = mm(sm, gi)
    m3 = mm(cps, gr + gi)
    tr = m1 - m2
    ti = m3 - m1 - m2
    n1 = mm(tr, cm)
    n2 = mm(ti, sm)
    n3 = mm(tr + ti, cps)
    out_ref[0, 0, 0] = n1 - n2
    out_ref[0, 0, 1] = n3 - n1 - n2


def _tc_call(ksp, k5_parts, cm, sm, cps):
    return pl.pallas_call(
        _tc_body,
        grid=(_BATCH,),
        in_specs=[
            pl.BlockSpec((1, 2, _RES, _RES), lambda b: (b, 0, 0, 0)),
            pl.BlockSpec((_NTEAM, 5, _RES, _RES), lambda b: (0, 0, 0, 0)),
            pl.BlockSpec((_RES, _RES), lambda b: (0, 0)),
            pl.BlockSpec((_RES, _RES), lambda b: (0, 0)),
            pl.BlockSpec((_RES, _RES), lambda b: (0, 0)),
        ],
        out_specs=pl.BlockSpec((1, 1, 2, _RES, _RES), lambda b: (b, 0, 0, 0, 0)),
        out_shape=jax.ShapeDtypeStruct((_BATCH, 1, 2, _RES, _RES), jnp.float32),
        scratch_shapes=[pltpu.VMEM((5, _RES, _RES), jnp.float32)],
        compiler_params=pltpu.CompilerParams(
            dimension_semantics=("arbitrary",),
        ),
    )(ksp, k5_parts, cm, sm, cps)


def kernel(k_space_input, trajectory):
    ksp = jnp.transpose(k_space_input, (0, 1, 4, 2, 3))[:, 0]

    traj_re = (trajectory.reshape(16, _NPTS // 16, 2)
               .transpose(2, 1, 0)
               .reshape(2, _NPTS))
    k5_parts = _k5_call(traj_re)
    k5_parts = k5_parts.reshape(_NTEAM, 5, _RES, _RES)

    cm = jnp.asarray(_C_NP, jnp.float32)
    sm = jnp.asarray(_S_NP, jnp.float32)
    cps = jnp.asarray(_CPS_NP, jnp.float32)

    return _tc_call(ksp, k5_parts, cm, sm, cps)
